# Initial kernel scaffold; baseline (speedup 1.0000x reference)
import jax
import jax.numpy as jnp
from jax.experimental import pallas as pl


def kernel(xyz):
    raise NotImplementedError("write your pallas kernel here")


# speedup vs baseline: 9.6820x; 9.6820x over previous
import jax
import jax.numpy as jnp
from jax.experimental import pallas as pl

K = 16
BQ = 256
N = 2048
BIG = 3.0e38


def _knn_body(x_ref, yt_ref, o_ref):
    x = x_ref[0]
    yt = yt_ref[0]
    xx = jnp.sum(x * x, axis=1, keepdims=True)
    yy = jnp.sum(yt * yt, axis=0, keepdims=True)
    inner = -2.0 * jax.lax.dot_general(
        x, yt, (((1,), (0,)), ((), ())),
        preferred_element_type=jnp.float32)
    d = (xx + inner) + yy
    col = jax.lax.broadcasted_iota(jnp.int32, (BQ, N), 1)
    cols = []
    for _ in range(K):
        m = jnp.min(d, axis=1, keepdims=True)
        j = jnp.min(jnp.where(d == m, col, N), axis=1, keepdims=True)
        cols.append(j)
        d = jnp.where(col == j, BIG, d)
    o_ref[0] = jnp.concatenate(cols, axis=1).astype(jnp.int32)


def kernel(xyz):
    B, n, _ = xyz.shape
    yt = jnp.transpose(xyz, (0, 2, 1))
    grid = (B, n // BQ)
    return pl.pallas_call(
        _knn_body,
        grid=grid,
        in_specs=[
            pl.BlockSpec((1, BQ, 3), lambda b, q: (b, q, 0)),
            pl.BlockSpec((1, 3, n), lambda b, q: (b, 0, 0)),
        ],
        out_specs=pl.BlockSpec((1, BQ, K), lambda b, q: (b, q, 0)),
        out_shape=jax.ShapeDtypeStruct((B, n, K), jnp.int32),
    )(xyz, yt)


# speedup vs baseline: 12.0051x; 1.2399x over previous
import jax
import jax.numpy as jnp
from jax.experimental import pallas as pl

K = 16
BQ = 256
N = 2048
BIG = 3.0e38


def _knn_body(x_ref, yt_ref, o_ref):
    x = x_ref[0]
    yt = yt_ref[0]
    xx = jnp.sum(x * x, axis=1, keepdims=True)
    yy = jnp.sum(yt * yt, axis=0, keepdims=True)
    inner = -2.0 * jax.lax.dot_general(
        x, yt, (((1,), (0,)), ((), ())),
        preferred_element_type=jnp.float32)
    d = (xx + inner) + yy
    col = jax.lax.broadcasted_iota(jnp.int32, (BQ, N), 1)
    cols = []
    for _ in range(K):
        j = jnp.argmin(d, axis=1).astype(jnp.int32)[:, None]
        cols.append(j)
        d = jnp.where(col == j, BIG, d)
    o_ref[0] = jnp.concatenate(cols, axis=1)


def kernel(xyz):
    B, n, _ = xyz.shape
    yt = jnp.transpose(xyz, (0, 2, 1))
    grid = (B, n // BQ)
    return pl.pallas_call(
        _knn_body,
        grid=grid,
        in_specs=[
            pl.BlockSpec((1, BQ, 3), lambda b, q: (b, q, 0)),
            pl.BlockSpec((1, 3, n), lambda b, q: (b, 0, 0)),
        ],
        out_specs=pl.BlockSpec((1, BQ, K), lambda b, q: (b, q, 0)),
        out_shape=jax.ShapeDtypeStruct((B, n, K), jnp.int32),
    )(xyz, yt)
